# baseline (speedup 1.0000x reference)
import functools

import jax
import jax.numpy as jnp
from jax import lax
from jax.experimental import pallas as pl
from jax.experimental.pallas import tpu as pltpu
from jax.experimental.pallas import tpu_sc as plsc

N_NODES = 10000
N_EDGES = 320000
D = 128
DA = 144
NC, NS = 2, 16
NW = NC * NS
C = 64
CHUNKS = 160
EPW = CHUNKS * C
E_PAD = EPW * NW
N_PAD = 10016
RPT = N_PAD // NS


def _sc_segment_accumulate(x_aug, src2, dst2):
  mesh = plsc.VectorSubcoreMesh(core_axis_name="c", subcore_axis_name="s")

  @functools.partial(
      pl.kernel,
      out_type=jax.ShapeDtypeStruct((NC, N_PAD, DA), jnp.float32),
      mesh=mesh,
      compiler_params=pltpu.CompilerParams(use_tc_tiling_on_sc=False),
      scratch_types=[
          pltpu.VMEM((CHUNKS, C), jnp.int32),
          pltpu.VMEM((CHUNKS, C), jnp.int32),
          pltpu.VMEM((2, C, DA), jnp.float32),
          pltpu.VMEM_SHARED((N_PAD, DA), jnp.float32),
          pltpu.SemaphoreType.DMA,
          pltpu.SemaphoreType.DMA,
          pltpu.SemaphoreType.DMA,
          pltpu.SemaphoreType.DMA,
      ],
  )
  def sc_fn(x_hbm, src_hbm, dst_hbm, out_hbm, idxs_v, idxd_v, rows_v, agg_sh,
            gsem0, gsem1, ssem0, ssem1):
    cid = lax.axis_index("c")
    sid = lax.axis_index("s")
    wid = sid * NC + cid

    zeros16 = jnp.zeros((16,), jnp.float32)

    def zero_row(i, _):
      for j in range(DA // 16):
        rows_v[0, i, j * 16:(j + 1) * 16] = zeros16
        rows_v[1, i, j * 16:(j + 1) * 16] = zeros16
      return 0

    lax.fori_loop(0, C, zero_row, 0)
    tr0 = sid * RPT
    for k in range(RPT // C):
      pltpu.sync_copy(rows_v.at[0], agg_sh.at[pl.ds(tr0 + k * C, C)])
    rem = RPT % C
    if rem:
      pltpu.sync_copy(rows_v.at[0].at[pl.ds(0, rem)],
                      agg_sh.at[pl.ds(tr0 + (RPT // C) * C, rem)])
    plsc.subcore_barrier()

    rbase = wid * CHUNKS
    pltpu.sync_copy(src_hbm.at[pl.ds(rbase, CHUNKS)], idxs_v)
    pltpu.sync_copy(dst_hbm.at[pl.ds(rbase, CHUNKS)], idxd_v)

    def gather(g, buf, sem):
      pltpu.async_copy(x_hbm.at[idxs_v.at[g]], rows_v.at[buf], sem)

    def gather_wait(buf, sem):
      pltpu.make_async_copy(x_hbm.at[idxs_v.at[0]], rows_v.at[buf], sem).wait()

    def scatter(g, buf, sem):
      pltpu.async_copy(rows_v.at[buf], agg_sh.at[idxd_v.at[g]], sem, add=True)

    def scatter_wait(buf, sem):
      pltpu.make_async_copy(rows_v.at[buf], agg_sh.at[pl.ds(0, C)], sem).wait()

    scatter(0, 1, ssem1)
    gather(0, 0, gsem0)

    def pair(p, _):
      g0 = 2 * p
      g1 = g0 + 1
      g2 = lax.rem(g0 + 2, CHUNKS)
      gather_wait(0, gsem0)
      scatter(g0, 0, ssem0)
      scatter_wait(1, ssem1)
      gather(g1, 1, gsem1)
      gather_wait(1, gsem1)
      scatter(g1, 1, ssem1)
      scatter_wait(0, ssem0)
      gather(g2, 0, gsem0)
      return 0

    lax.fori_loop(0, CHUNKS // 2, pair, 0)
    gather_wait(0, gsem0)
    scatter_wait(1, ssem1)
    plsc.subcore_barrier()

    pltpu.sync_copy(agg_sh.at[pl.ds(tr0, RPT)],
                    out_hbm.at[cid].at[pl.ds(tr0, RPT)])

  return sc_fn(x_aug, src2, dst2)


def _tc_finish(partials, x, W, b2):
  BR = 400
  grid = (N_NODES // BR,)

  def tc_fn(p_ref, x_ref, w_ref, b_ref, o_ref):
    p = p_ref[...]
    s = p[0] + p[1]
    agg = s[:, :D]
    deg = jnp.maximum(s[:, D:D + 1], 1.0)
    h = jnp.dot(agg / deg + x_ref[...], w_ref[...],
                preferred_element_type=jnp.float32)
    o_ref[...] = jnp.maximum(h + b_ref[...], 0.0)

  return pl.pallas_call(
      tc_fn,
      grid=grid,
      in_specs=[
          pl.BlockSpec((NC, BR, DA), lambda i: (0, i, 0)),
          pl.BlockSpec((BR, D), lambda i: (i, 0)),
          pl.BlockSpec((D, D), lambda i: (0, 0)),
          pl.BlockSpec((1, D), lambda i: (0, 0)),
      ],
      out_specs=pl.BlockSpec((BR, D), lambda i: (i, 0)),
      out_shape=jax.ShapeDtypeStruct((N_NODES, D), jnp.float32),
  )(partials, x, W, b2)


@jax.jit
def kernel(x, edge_index, W, b):
  src = edge_index[0]
  dst = edge_index[1]
  pad = E_PAD - N_EDGES
  src2 = jnp.concatenate([src, jnp.zeros((pad,), jnp.int32)]).reshape(
      NW * CHUNKS, C)
  dst2 = jnp.concatenate([dst, jnp.full((pad,), N_NODES, jnp.int32)]).reshape(
      NW * CHUNKS, C)
  ones_col = jnp.ones((N_NODES, 1), jnp.float32)
  zpad = jnp.zeros((N_NODES, DA - D - 1), jnp.float32)
  x_aug = jnp.concatenate([x, ones_col, zpad], axis=1)

  partials = _sc_segment_accumulate(x_aug, src2, dst2)
  return _tc_finish(partials, x, W, b.reshape(1, D))


# speedup vs baseline: 1.0943x; 1.0943x over previous
import functools

import jax
import jax.numpy as jnp
from jax import lax
from jax.experimental import pallas as pl
from jax.experimental.pallas import tpu as pltpu
from jax.experimental.pallas import tpu_sc as plsc

N_NODES = 10000
N_EDGES = 320000
D = 128
DA = 144
NC, NS = 2, 16
NW = NC * NS
C = 128
CHUNKS = 80
BLK = 8
NBLK = CHUNKS // BLK
EPW = CHUNKS * C
E_PAD = EPW * NW
RPT = N_NODES // NS


def _sc_segment_accumulate(x_aug, src2, dst2):
  mesh = plsc.VectorSubcoreMesh(core_axis_name="c", subcore_axis_name="s")

  @functools.partial(
      pl.kernel,
      out_type=jax.ShapeDtypeStruct((NC, N_NODES, DA), jnp.float32),
      mesh=mesh,
      compiler_params=pltpu.CompilerParams(use_tc_tiling_on_sc=False),
      scratch_types=[
          pltpu.VMEM((2, BLK, C), jnp.int32),
          pltpu.VMEM((2, BLK, C), jnp.int32),
          pltpu.VMEM((2, C, DA), jnp.float32),
          pltpu.VMEM_SHARED((N_NODES, DA), jnp.float32),
          pltpu.SemaphoreType.DMA,
          pltpu.SemaphoreType.DMA,
          pltpu.SemaphoreType.DMA,
          pltpu.SemaphoreType.DMA,
          pltpu.SemaphoreType.DMA,
      ],
  )
  def sc_fn(x_hbm, src_hbm, dst_hbm, out_hbm, idxs_v, idxd_v, rows_v, agg_sh,
            gsem0, gsem1, ssem0, ssem1, isem):
    cid = lax.axis_index("c")
    sid = lax.axis_index("s")
    wid = sid * NC + cid

    zeros16 = jnp.zeros((16,), jnp.float32)

    def zero_row(i, _):
      for j in range(DA // 16):
        rows_v[0, i, j * 16:(j + 1) * 16] = zeros16
        rows_v[1, i, j * 16:(j + 1) * 16] = zeros16
      return 0

    lax.fori_loop(0, C, zero_row, 0)
    tr0 = sid * RPT
    for k in range(RPT // C):
      pltpu.sync_copy(rows_v.at[0], agg_sh.at[pl.ds(tr0 + k * C, C)])
    rem = RPT % C
    if rem:
      pltpu.sync_copy(rows_v.at[0].at[pl.ds(0, rem)],
                      agg_sh.at[pl.ds(tr0 + (RPT // C) * C, rem)])
    plsc.subcore_barrier()

    rbase = wid * CHUNKS

    def idx_fetch(blk, buf):
      r = rbase + blk * BLK
      pltpu.async_copy(src_hbm.at[pl.ds(r, BLK)], idxs_v.at[buf], isem)
      pltpu.async_copy(dst_hbm.at[pl.ds(r, BLK)], idxd_v.at[buf], isem)

    def idx_wait(buf):
      pltpu.make_async_copy(src_hbm.at[pl.ds(0, BLK)], idxs_v.at[buf],
                            isem).wait()
      pltpu.make_async_copy(dst_hbm.at[pl.ds(0, BLK)], idxd_v.at[buf],
                            isem).wait()

    def gather(pb, j, buf, sem):
      pltpu.async_copy(x_hbm.at[idxs_v.at[pb, j]], rows_v.at[buf], sem)

    def gather_wait(buf, sem):
      pltpu.make_async_copy(x_hbm.at[idxs_v.at[0, 0]], rows_v.at[buf],
                            sem).wait()

    def scatter(pb, j, buf, sem):
      pltpu.async_copy(rows_v.at[buf], agg_sh.at[idxd_v.at[pb, j]], sem,
                       add=True)

    def scatter_wait(buf, sem):
      pltpu.make_async_copy(rows_v.at[buf], agg_sh.at[pl.ds(0, C)],
                            sem).wait()

    idx_fetch(0, 0)
    idx_wait(0)
    scatter(0, 0, 1, ssem1)
    gather(0, 0, 0, gsem0)

    def block(b, _):
      pb = lax.rem(b, 2)
      pn = 1 - pb
      nblk = lax.rem(b + 1, NBLK)
      for q in range(BLK // 2):
        j0, j1 = 2 * q, 2 * q + 1
        gather_wait(0, gsem0)
        scatter(pb, j0, 0, ssem0)
        scatter_wait(1, ssem1)
        if q == 0:
          idx_fetch(nblk, pn)
        gather(pb, j1, 1, gsem1)
        gather_wait(1, gsem1)
        scatter(pb, j1, 1, ssem1)
        scatter_wait(0, ssem0)
        if q == BLK // 2 - 1:
          idx_wait(pn)
          gather(pn, 0, 0, gsem0)
        else:
          gather(pb, j1 + 1, 0, gsem0)
      return 0

    lax.fori_loop(0, NBLK, block, 0)
    gather_wait(0, gsem0)
    scatter_wait(1, ssem1)
    plsc.subcore_barrier()

    pltpu.sync_copy(agg_sh.at[pl.ds(tr0, RPT)],
                    out_hbm.at[cid].at[pl.ds(tr0, RPT)])

  return sc_fn(x_aug, src2, dst2)


def _tc_finish(partials, x, W, b2):
  BR = 400
  grid = (N_NODES // BR,)

  def tc_fn(p_ref, x_ref, w_ref, b_ref, o_ref):
    p = p_ref[...]
    s = p[0] + p[1]
    agg = s[:, :D]
    deg = jnp.maximum(s[:, D:D + 1], 1.0)
    h = jnp.dot(agg / deg + x_ref[...], w_ref[...],
                preferred_element_type=jnp.float32)
    o_ref[...] = jnp.maximum(h + b_ref[...], 0.0)

  return pl.pallas_call(
      tc_fn,
      grid=grid,
      in_specs=[
          pl.BlockSpec((NC, BR, DA), lambda i: (0, i, 0)),
          pl.BlockSpec((BR, D), lambda i: (i, 0)),
          pl.BlockSpec((D, D), lambda i: (0, 0)),
          pl.BlockSpec((1, D), lambda i: (0, 0)),
      ],
      out_specs=pl.BlockSpec((BR, D), lambda i: (i, 0)),
      out_shape=jax.ShapeDtypeStruct((N_NODES, D), jnp.float32),
  )(partials, x, W, b2)


@jax.jit
def kernel(x, edge_index, W, b):
  src = edge_index[0]
  dst = edge_index[1]
  pad = E_PAD - N_EDGES
  src2 = jnp.concatenate([src, jnp.full((pad,), N_NODES, jnp.int32)]).reshape(
      NW * CHUNKS, C)
  dst2 = jnp.concatenate([dst, jnp.zeros((pad,), jnp.int32)]).reshape(
      NW * CHUNKS, C)
  ones_col = jnp.ones((N_NODES, 1), jnp.float32)
  zpad = jnp.zeros((N_NODES, DA - D - 1), jnp.float32)
  x_aug = jnp.concatenate([x, ones_col, zpad], axis=1)
  x_aug = jnp.concatenate([x_aug, jnp.zeros((1, DA), jnp.float32)], axis=0)

  partials = _sc_segment_accumulate(x_aug, src2, dst2)
  return _tc_finish(partials, x, W, b.reshape(1, D))
